# baseline (speedup 1.0000x reference)
import functools
import math

import jax
import jax.numpy as jnp
from jax import lax
from jax.experimental import pallas as pl
from jax.experimental.pallas import tpu as pltpu
from jax.experimental.pallas import tpu_sc as plsc

D_MODEL = 64
LANES = 16
GATHER_ROWS = 128


@functools.partial(jax.jit, static_argnums=(0, 1, 2))
def _run(num_idx, num_cores, num_subcores, idx2d, table):
  nw = num_cores * num_subcores
  b_per_w = num_idx // nw
  chunk = 1024
  ng = chunk // GATHER_ROWS
  n_chunks = b_per_w // chunk
  idx_rows_per_w = b_per_w // GATHER_ROWS
  scale = float(math.sqrt(D_MODEL))

  mesh = plsc.VectorSubcoreMesh(core_axis_name="c", subcore_axis_name="s")

  @functools.partial(
      pl.kernel,
      mesh=mesh,
      out_type=jax.ShapeDtypeStruct((num_idx, D_MODEL), jnp.float32),
      scratch_types=[
          pltpu.VMEM((ng, GATHER_ROWS), jnp.int32),
          pltpu.VMEM((chunk, D_MODEL), jnp.float32),
          pltpu.SemaphoreType.DMA,
      ],
      compiler_params=pltpu.CompilerParams(use_tc_tiling_on_sc=False),
  )
  def emb_kernel(idx_hbm, table_hbm, out_hbm, idx_v, rows_v, sem):
    wid = lax.axis_index("s") * num_cores + lax.axis_index("c")
    idx_row_base = wid * idx_rows_per_w
    out_base = wid * b_per_w

    def chunk_body(g, carry):
      pltpu.sync_copy(idx_hbm.at[pl.ds(idx_row_base + g * ng, ng)], idx_v)
      handles = []
      for i in range(ng):
        handles.append(
            pltpu.async_copy(
                table_hbm.at[idx_v.at[i]],
                rows_v.at[pl.ds(i * GATHER_ROWS, GATHER_ROWS)],
                sem,
            ))
      for h in handles:
        h.wait()

      def row_body(j, c):
        for t in range(D_MODEL // LANES):
          sl = pl.ds(t * LANES, LANES)
          rows_v[j, sl] = rows_v[j, sl] * scale
        return c

      lax.fori_loop(0, chunk, row_body, 0)
      pltpu.sync_copy(rows_v, out_hbm.at[pl.ds(out_base + g * chunk, chunk)])
      return carry

    lax.fori_loop(0, n_chunks, chunk_body, 0)

  return emb_kernel(idx2d, table)


def kernel(src, table):
  num_idx = src.size
  info = plsc.get_sparse_core_info()
  idx2d = src.reshape(num_idx // GATHER_ROWS, GATHER_ROWS).astype(jnp.int32)
  out = _run(num_idx, info.num_cores, info.num_subcores, idx2d, table)
  return out.reshape(*src.shape, D_MODEL)
